# Initial kernel scaffold; baseline (speedup 1.0000x reference)
import jax
import jax.numpy as jnp
from jax.experimental import pallas as pl


def kernel(x, edge_index, edge_type, proj_W, proj_b, bases0, comp0, bases1, comp1, root1, bases2, comp2, root2):
    raise NotImplementedError("write your pallas kernel here")


# speedup vs baseline: 11.9430x; 11.9430x over previous
import functools

import jax
import jax.numpy as jnp
from jax import lax
from jax.experimental import pallas as pl
from jax.experimental.pallas import tpu as pltpu
from jax.experimental.pallas import tpu_sc as plsc

N = 10000
E = 320000
D = 128
R = 4
NB = 8

CB = 4
CW = D // CB
NS = 16
EPT = E // NS
CHW = 128
NCH = -(-EPT // CHW)
EPT_PAD = NCH * CHW
DUMMY = R * N
ACC_ROWS = R * N + 8
RPT = 2496
RTAIL = R * N - NS * RPT
BN = 1000
CNTW = 8


def _edge_prep_body(src_ref, dst_ref, et_ref, ci_ref, src4_ref):
    ci_ref[...] = et_ref[...] * N + dst_ref[...]
    for cb in range(CB):
        src4_ref[cb] = src_ref[...] + cb * N


def _edge_prep(src_p, dst_p, et_p):
    return pl.pallas_call(
        _edge_prep_body,
        out_shape=(
            jax.ShapeDtypeStruct((NS, NCH, CHW), jnp.int32),
            jax.ShapeDtypeStruct((CB, NS, NCH, CHW), jnp.int32),
        ),
    )(src_p, dst_p, et_p)


def _weights_body(comp_ref, bases_ref, w_ref):
    w_ref[...] = jnp.dot(comp_ref[...], bases_ref[...],
                         preferred_element_type=jnp.float32)


def _make_weights(comp, bases):
    w = pl.pallas_call(
        _weights_body,
        out_shape=jax.ShapeDtypeStruct((R, D * D), jnp.float32),
    )(comp, bases.reshape(NB, D * D))
    return w.reshape(R, D, D)


def _proj_body(x_ref, w_ref, b_ref, h_ref, hblk_ref):
    h = jnp.dot(x_ref[...], w_ref[...], preferred_element_type=jnp.float32)
    h = jnp.maximum(h + b_ref[...], 0.0)
    h_ref[...] = h
    for cb in range(CB):
        hblk_ref[cb] = h[:, cb * CW:(cb + 1) * CW]


def _proj(x, w, b):
    return pl.pallas_call(
        _proj_body,
        grid=(N // BN,),
        in_specs=[
            pl.BlockSpec((BN, D), lambda i: (i, 0)),
            pl.BlockSpec((D, D), lambda i: (0, 0)),
            pl.BlockSpec((1, D), lambda i: (0, 0)),
        ],
        out_specs=(
            pl.BlockSpec((BN, D), lambda i: (i, 0)),
            pl.BlockSpec((CB, BN, CW), lambda i: (0, i, 0)),
        ),
        out_shape=(
            jax.ShapeDtypeStruct((N, D), jnp.float32),
            jax.ShapeDtypeStruct((CB, N, CW), jnp.float32),
        ),
    )(x, w, b)


def _layer_body(sums_ref, cnt_ref, w_ref, hprev_ref, root_ref,
                h_ref, hblk_ref, *, relu, want_blk):
    rec = 1.0 / jnp.maximum(cnt_ref[...], 1.0)
    acc = jnp.zeros((BN, D), jnp.float32)
    for r in range(R):
        rr = rec[r, :, 0:1]
        for cb in range(CB):
            acc = acc + jnp.dot(sums_ref[cb, r] * rr,
                                w_ref[r, cb * CW:(cb + 1) * CW, :],
                                preferred_element_type=jnp.float32)
    if root_ref is not None:
        acc = acc + jnp.dot(hprev_ref[...], root_ref[...],
                            preferred_element_type=jnp.float32)
    if relu:
        acc = jnp.maximum(acc, 0.0)
    h_ref[...] = acc
    if want_blk:
        for cb in range(CB):
            hblk_ref[cb] = acc[:, cb * CW:(cb + 1) * CW]


def _layer_tc(sums4, counts, w, hprev, root, relu, want_blk):
    in_specs = [
        pl.BlockSpec((CB, R, BN, CW), lambda i: (0, 0, i, 0)),
        pl.BlockSpec((R, BN, CNTW), lambda i: (0, i, 0)),
        pl.BlockSpec((R, D, D), lambda i: (0, 0, 0)),
    ]
    args = [sums4, counts, w]
    if root is not None:
        in_specs.append(pl.BlockSpec((BN, D), lambda i: (i, 0)))
        in_specs.append(pl.BlockSpec((D, D), lambda i: (0, 0)))
        args.append(hprev)
        args.append(root)

    out_specs = [pl.BlockSpec((BN, D), lambda i: (i, 0))]
    out_shape = [jax.ShapeDtypeStruct((N, D), jnp.float32)]
    if want_blk:
        out_specs.append(pl.BlockSpec((CB, BN, CW), lambda i: (0, i, 0)))
        out_shape.append(jax.ShapeDtypeStruct((CB, N, CW), jnp.float32))

    def body(*refs):
        if root is not None:
            sums_ref, cnt_ref, w_ref, hp_ref, rt_ref = refs[:5]
            orefs = refs[5:]
        else:
            sums_ref, cnt_ref, w_ref = refs[:3]
            hp_ref = rt_ref = None
            orefs = refs[3:]
        h_ref = orefs[0]
        hblk_ref = orefs[1] if want_blk else None
        _layer_body(sums_ref, cnt_ref, w_ref, hp_ref, rt_ref,
                    h_ref, hblk_ref, relu=relu, want_blk=want_blk)

    res = pl.pallas_call(
        body,
        grid=(N // BN,),
        in_specs=in_specs,
        out_specs=tuple(out_specs) if want_blk else out_specs[0],
        out_shape=tuple(out_shape) if want_blk else out_shape[0],
    )(*args)
    return res if want_blk else (res, None)


def _make_agg():
    mesh = plsc.VectorSubcoreMesh(core_axis_name="c", subcore_axis_name="s")

    scratch = [
        pltpu.VMEM((NCH, CHW), jnp.int32),
        pltpu.VMEM((NCH, CHW), jnp.int32),
        pltpu.VMEM((CHW, CW), jnp.float32),
        pltpu.VMEM((CHW, CW), jnp.float32),
        pltpu.VMEM_SHARED((ACC_ROWS, CW), jnp.float32),
        pltpu.SemaphoreType.DMA,
        pltpu.SemaphoreType.DMA,
    ]

    def body(hflat, src4, ci3, zrows, sums,
             src_all, ci_all, rowsA, rowsB, acc, semA, semB):
        c = lax.axis_index("c")
        s = lax.axis_index("s")

        pltpu.sync_copy(ci3.at[s], ci_all)

        for p in range(2):
            cb = 2 * c + p

            pltpu.sync_copy(src4.at[cb, s], src_all)
            pltpu.sync_copy(zrows, acc.at[pl.ds(s * RPT, RPT)])

            @pl.when(s < 8)
            def _():
                pltpu.sync_copy(zrows.at[pl.ds(0, 8)],
                                acc.at[pl.ds(NS * RPT + s * 8, 8)])

            plsc.subcore_barrier()

            def gather(idx, buf, sem):
                return pltpu.make_async_copy(hflat.at[src_all.at[idx]],
                                             buf, sem)

            def scatter(idx, buf):
                pltpu.sync_copy(buf, acc.at[ci_all.at[idx]], add=True)

            gather(0, rowsA, semA).start()

            def loop_body(j, carry):
                a = 2 * j
                b = a + 1
                gather(b, rowsB, semB).start()
                gather(a, rowsA, semA).wait()
                scatter(a, rowsA)
                gather(a + 2, rowsA, semA).start()
                gather(b, rowsB, semB).wait()
                scatter(b, rowsB)
                return carry

            lax.fori_loop(0, (NCH - 1) // 2, loop_body, 0)
            last = NCH - 1
            gather(last, rowsA, semA).wait()
            scatter(last, rowsA)

            plsc.subcore_barrier()
            pltpu.sync_copy(acc.at[pl.ds(s * RPT, RPT)],
                            sums.at[cb, pl.ds(s * RPT, RPT)])

            @pl.when(s < 8)
            def _():
                pltpu.sync_copy(acc.at[pl.ds(NS * RPT + s * 8, 8)],
                                sums.at[cb, pl.ds(NS * RPT + s * 8, 8)])

            plsc.subcore_barrier()

    return pl.kernel(
        body,
        out_type=jax.ShapeDtypeStruct((CB, R * N, CW), jnp.float32),
        mesh=mesh,
        scratch_types=scratch,
        compiler_params=pltpu.CompilerParams(use_tc_tiling_on_sc=False),
    )


def _make_count():
    mesh = plsc.VectorSubcoreMesh(core_axis_name="c", subcore_axis_name="s")

    scratch = [
        pltpu.VMEM((NCH, CHW), jnp.int32),
        pltpu.VMEM((CHW, CNTW), jnp.float32),
        pltpu.VMEM_SHARED((ACC_ROWS, CNTW), jnp.float32),
        pltpu.SemaphoreType.DMA,
    ]

    def body(ci3, zrows_c, ones_hbm, counts, ci_all, ones_v, acc_c, sem):
        c = lax.axis_index("c")
        s = lax.axis_index("s")

        @pl.when(c == 0)
        def _():
            pltpu.sync_copy(ci3.at[s], ci_all)
            pltpu.sync_copy(ones_hbm, ones_v)
            pltpu.sync_copy(zrows_c, acc_c.at[pl.ds(s * RPT, RPT)])

            @pl.when(s < 8)
            def _():
                pltpu.sync_copy(zrows_c.at[pl.ds(0, 8)],
                                acc_c.at[pl.ds(NS * RPT + s * 8, 8)])

            plsc.subcore_barrier()

            def loop_body(j, carry):
                pltpu.sync_copy(ones_v, acc_c.at[ci_all.at[j]], add=True)
                return carry

            lax.fori_loop(0, NCH, loop_body, 0)

            plsc.subcore_barrier()
            pltpu.sync_copy(acc_c.at[pl.ds(s * RPT, RPT)],
                            counts.at[pl.ds(s * RPT, RPT)])

            @pl.when(s < 8)
            def _():
                pltpu.sync_copy(acc_c.at[pl.ds(NS * RPT + s * 8, 8)],
                                counts.at[pl.ds(NS * RPT + s * 8, 8)])

    return pl.kernel(
        body,
        out_type=jax.ShapeDtypeStruct((R * N, CNTW), jnp.float32),
        mesh=mesh,
        scratch_types=scratch,
        compiler_params=pltpu.CompilerParams(use_tc_tiling_on_sc=False),
    )


_agg = _make_agg()
_count = _make_count()


def kernel(x, edge_index, edge_type, proj_W, proj_b,
           bases0, comp0, bases1, comp1, root1, bases2, comp2, root2):
    src = edge_index[0]
    dst = edge_index[1]

    pad = EPT_PAD - EPT
    src_p = jnp.pad(src.reshape(NS, EPT), ((0, 0), (0, pad))) \
        .reshape(NS, NCH, CHW)
    dst_p = jnp.pad(dst.reshape(NS, EPT), ((0, 0), (0, pad))) \
        .reshape(NS, NCH, CHW)
    et_p = jnp.pad(edge_type.reshape(NS, EPT), ((0, 0), (0, pad)),
                   constant_values=R).reshape(NS, NCH, CHW)

    ci3, src4 = _edge_prep(src_p, dst_p, et_p)

    w0 = _make_weights(comp0, bases0)
    w1 = _make_weights(comp1, bases1)
    w2 = _make_weights(comp2, bases2)

    h0, h0blk = _proj(x, proj_W, proj_b.reshape(1, D))

    zrows = jnp.zeros((RPT, CW), jnp.float32)
    zrows_c = jnp.zeros((RPT, CNTW), jnp.float32)
    ones8 = jnp.ones((CHW, CNTW), jnp.float32)

    counts = _count(ci3, zrows_c, ones8)
    counts = counts.reshape(R, N, CNTW)

    sums0 = _agg(h0blk.reshape(CB * N, CW), src4, ci3, zrows)
    sums0 = sums0.reshape(CB, R, N, CW)

    h1, h1blk = _layer_tc(sums0, counts, w0, None, None,
                          relu=True, want_blk=True)

    sums1 = _agg(h1blk.reshape(CB * N, CW), src4, ci3, zrows)
    sums1 = sums1.reshape(CB, R, N, CW)

    h2, h2blk = _layer_tc(sums1, counts, w1, h1, root1,
                          relu=True, want_blk=True)

    sums2 = _agg(h2blk.reshape(CB * N, CW), src4, ci3, zrows)
    sums2 = sums2.reshape(CB, R, N, CW)

    out, _ = _layer_tc(sums2, counts, w2, h2, root2,
                       relu=False, want_blk=False)

    return (out, h0, h1, h2)
